# Initial kernel scaffold; baseline (speedup 1.0000x reference)
import jax
import jax.numpy as jnp
from jax.experimental import pallas as pl


def kernel(R, Z, N, NM, emb, W_rbf, b_rbf, Wa1, ba1, Wa2, ba2, Wt1, bt1, Wt2, bt2, Wt3, bt3, Wq1, bq1, Wq2, bq2, Wq3, bq3):
    raise NotImplementedError("write your pallas kernel here")


# speedup vs baseline: 3.9823x; 3.9823x over previous
import functools

import jax
import jax.numpy as jnp
from jax import lax
from jax.experimental import pallas as pl
from jax.experimental.pallas import tpu as pltpu

_B, _A, _NB = 8, 256, 64
_NF = 128
_RES = 20
_CUT = 5.0
_TA = 16
_E = _TA * _NB


def _silu_fb(y):
    s = jax.nn.sigmoid(y)
    return y * s, s * (1.0 + y * (1.0 - s))


def _prep_kernel(z_ref, emb_ref, wa1_ref, ba1_ref, wa2_ref, ba2_ref,
                 a0_ref, am_ref):
    z = z_ref[...]
    oh = (z == lax.broadcasted_iota(jnp.int32, (_B * _A, 16), 1)
          ).astype(jnp.float32)
    a0 = lax.dot(oh, emb_ref[...], preferred_element_type=jnp.float32,
                 precision=lax.Precision.HIGHEST)
    y = lax.dot(a0, wa1_ref[...], preferred_element_type=jnp.float32) + ba1_ref[...]
    h, _ = _silu_fb(y)
    am = lax.dot(h, wa2_ref[...], preferred_element_type=jnp.float32) + ba2_ref[...]
    a0_ref[...] = a0
    am_ref[...] = am


def _edge_kernel(r_ref, n_ref, nm_ref, a0_ref, am_ref,
                 wrbf_ref, wrbft_ref, brbf_ref,
                 wt1_ref, bt1_ref, wt2_ref, bt2_ref, wt3t_ref, bt3_ref,
                 wt1t_ref, wt2t_ref,
                 wq1_ref, bq1_ref, wq2_ref, bq2_ref, wq3t_ref, bq3_ref,
                 wq1t_ref, wq2t_ref,
                 ap_ref, pp_ref, a1_ref, p1_ref, gai_ref, gp_ref, d_ref, v_ref):
    ia = pl.program_id(1)
    i0 = ia * _TA
    c = _CUT
    sq2c = (2.0 / c) ** 0.5

    rb = r_ref[0]
    nf = n_ref[0]
    nm = nm_ref[0]
    a0t = a0_ref[0]
    amt = am_ref[0]

    oh = (nf == lax.broadcasted_iota(jnp.int32, (_E, _A), 1)
          ).astype(jnp.float32)
    eidx = i0 + lax.broadcasted_iota(jnp.int32, (_E, 1), 0) // _NB
    self_e = nf == eidx

    rj = lax.dot(oh, rb, preferred_element_type=jnp.float32,
                 precision=lax.Precision.HIGHEST)
    ri = r_ref[0, pl.ds(i0, _TA), :]
    ri_e = jnp.broadcast_to(ri[:, None, :], (_TA, _NB, 3)).reshape(_E, 3)
    vv = jnp.where(self_e, 0.0, rj - ri_e)
    s = jnp.sum(vv * vv, axis=-1, keepdims=True) + 1e-12
    dv = jnp.sqrt(s)
    dd = dv * nm

    x = dd / c
    ltc = (dd < c).astype(jnp.float32)
    x8 = (x * x) * (x * x)
    x8 = x8 * x8
    x9 = x8 * x
    pc = (1.0 - 55.0 * x9 + 99.0 * x9 * x - 45.0 * x9 * x * x) * ltc
    dpc = (1.0 / c) * (-495.0 * x8 + 990.0 * x9 - 495.0 * x9 * x) * ltc

    nrow = (lax.broadcasted_iota(jnp.int32, (_E, _RES), 1) + 1
            ).astype(jnp.float32)
    arg = dd * (jnp.pi / c) * nrow
    dn = dd + 1e-8
    rdn = 1.0 / dn
    sn = jnp.sin(arg)
    cs = jnp.cos(arg)
    rbf = sq2c * sn * rdn
    drbf = sq2c * rdn * (nrow * (jnp.pi / c) * cs - sn * rdn)

    rbf_d = lax.dot(rbf, wrbf_ref[...],
                    preferred_element_type=jnp.float32) + brbf_ref[...]
    rbf_m = rbf_d * pc

    a0i = a0_ref[0, pl.ds(i0, _TA), :]
    ami = am_ref[0, pl.ds(i0, _TA), :]
    a0i_e = jnp.broadcast_to(a0i[:, None, :], (_TA, _NB, _NF)).reshape(_E, _NF)
    ami_e = jnp.broadcast_to(ami[:, None, :], (_TA, _NB, _NF)).reshape(_E, _NF)
    a0j = lax.dot(oh, a0t, preferred_element_type=jnp.float32,
                  precision=lax.Precision.HIGHEST)
    amj = lax.dot(oh, amt, preferred_element_type=jnp.float32,
                  precision=lax.Precision.HIGHEST)

    am2 = ami_e * amj
    msij = am2 * rbf_m * nm
    p1v = a0i_e * a0j + msij
    a1v = a0i + jnp.sum(msij.reshape(_TA, _NB, _NF), axis=1)

    y1 = lax.dot(a1v, wt1_ref[...], preferred_element_type=jnp.float32) + bt1_ref[...]
    h1, d1 = _silu_fb(y1)
    y2 = lax.dot(h1, wt2_ref[...], preferred_element_type=jnp.float32) + bt2_ref[...]
    h2, d2 = _silu_fb(y2)
    wt3b = wt3t_ref[...].astype(jnp.bfloat16).astype(jnp.float32)
    h2b = h2.astype(jnp.bfloat16).astype(jnp.float32)
    apv = jnp.sum(h2b * wt3b, axis=-1, keepdims=True) + bt3_ref[...]
    g_y2 = wt3b * d2
    g_h1 = lax.dot(g_y2, wt2t_ref[...], preferred_element_type=jnp.float32)
    ga1 = lax.dot(g_h1 * d1, wt1t_ref[...],
                  preferred_element_type=jnp.float32)

    yq1 = lax.dot(p1v, wq1_ref[...], preferred_element_type=jnp.float32) + bq1_ref[...]
    q1, dq1 = _silu_fb(yq1)
    yq2 = lax.dot(q1, wq2_ref[...], preferred_element_type=jnp.float32) + bq2_ref[...]
    q2, dq2 = _silu_fb(yq2)
    wq3b = wq3t_ref[...].astype(jnp.bfloat16).astype(jnp.float32)
    q3 = jnp.sum(q2.astype(jnp.bfloat16).astype(jnp.float32) * wq3b,
                 axis=-1, keepdims=True) + bq3_ref[...]
    pp_sq = q3 * q3
    t = jnp.tanh(10.0 * (5.0 - dd))
    tap = 0.5 + 0.5 * t
    mask = (dd != 0.0).astype(jnp.float32)
    ppv = pp_sq * mask * tap

    g_q3 = 2.0 * q3 * mask * tap
    g_q3b = g_q3.astype(jnp.bfloat16).astype(jnp.float32)
    g_yq2 = (g_q3b * wq3b) * dq2
    g_q1 = lax.dot(g_yq2, wq2t_ref[...], preferred_element_type=jnp.float32)
    g_p1 = lax.dot(g_q1 * dq1, wq1t_ref[...],
                   preferred_element_type=jnp.float32)

    def chain_to_d(g_ms):
        g_rbf_m = g_ms * am2 * nm
        g_pc = jnp.sum(g_rbf_m * rbf_d, axis=-1, keepdims=True)
        g_rbf = lax.dot(g_rbf_m * pc, wrbft_ref[...],
                        preferred_element_type=jnp.float32)
        return (jnp.sum(g_rbf * drbf, axis=-1, keepdims=True)
                + g_pc * dpc)

    gd_pair = chain_to_d(g_p1) + pp_sq * mask * (-5.0 * (1.0 - t * t))

    ga1_e = jnp.broadcast_to(ga1[:, None, :], (_TA, _NB, _NF)).reshape(_E, _NF)
    gd_atom = chain_to_d(ga1_e)
    w = (gd_atom * nm / dv) * vv

    ap_ref[0] = apv
    pp_ref[0] = ppv
    a1_ref[0] = a1v
    p1_ref[0] = p1v
    gp_ref[0] = gd_pair
    d_ref[0] = dd
    v_ref[0] = vv

    ci = (eidx == lax.broadcasted_iota(jnp.int32, (_E, _A), 1)
          ).astype(jnp.float32)
    scat = lax.dot_general(oh - ci, w, (((0,), (0,)), ((), ())),
                           preferred_element_type=jnp.float32,
                           precision=lax.Precision.HIGHEST)

    @pl.when(ia == 0)
    def _():
        gai_ref[0] = jnp.zeros((_A, 3), dtype=jnp.float32)

    gai_ref[0] = gai_ref[0] + scat


@functools.partial(jax.jit, static_argnames=("interpret",))
def _run(R, Z, N, NM, emb, W_rbf, b_rbf, Wa1, ba1, Wa2, ba2,
         Wt1, bt1, Wt2, bt2, Wt3, bt3, Wq1, bq1, Wq2, bq2, Wq3, bq3,
         interpret=False):
    f32 = jnp.float32
    z2 = Z.reshape(_B * _A, 1).astype(jnp.int32)
    emb16 = jnp.zeros((16, _NF), f32).at[:10].set(emb)

    a0f, amf = pl.pallas_call(
        _prep_kernel,
        out_shape=(jax.ShapeDtypeStruct((_B * _A, _NF), f32),
                   jax.ShapeDtypeStruct((_B * _A, _NF), f32)),
        interpret=interpret,
    )(z2, emb16, Wa1, ba1.reshape(1, _NF), Wa2, ba2.reshape(1, _NF))

    a0 = a0f.reshape(_B, _A, _NF)
    am = amf.reshape(_B, _A, _NF)

    grid = (_B, _A // _TA)
    full = lambda *dims: pl.BlockSpec(dims, lambda b, i: (0,) * len(dims))
    batch = lambda *dims: pl.BlockSpec((1,) + dims, lambda b, i: (b,) + (0,) * len(dims))
    tile = lambda *dims: pl.BlockSpec((1, _TA) + dims,
                                      lambda b, i: (b, i) + (0,) * len(dims))

    AE = _A * _NB
    etile = lambda *dims: pl.BlockSpec((1, _E) + dims,
                                       lambda b, i: (b, i) + (0,) * len(dims))
    out_shapes = (
        jax.ShapeDtypeStruct((_B, _A, 1), f32),
        jax.ShapeDtypeStruct((_B, AE, 1), f32),
        jax.ShapeDtypeStruct((_B, _A, _NF), f32),
        jax.ShapeDtypeStruct((_B, AE, _NF), f32),
        jax.ShapeDtypeStruct((_B, _A, 3), f32),
        jax.ShapeDtypeStruct((_B, AE, 1), f32),
        jax.ShapeDtypeStruct((_B, AE, 1), f32),
        jax.ShapeDtypeStruct((_B, AE, 3), f32),
    )
    out_specs = (
        tile(1), etile(1), tile(_NF), etile(_NF),
        batch(_A, 3), etile(1), etile(1), etile(3),
    )
    in_specs = [
        batch(_A, 3),
        etile(1),
        etile(1),
        batch(_A, _NF),
        batch(_A, _NF),
        full(_RES, _NF),
        full(_NF, _RES),
        full(1, _NF),
        full(_NF, _NF), full(1, _NF),
        full(_NF, 64), full(1, 64),
        full(1, 64), full(1, 1),
        full(_NF, _NF), full(64, _NF),
        full(_NF, _NF), full(1, _NF),
        full(_NF, 64), full(1, 64),
        full(1, 64), full(1, 1),
        full(_NF, _NF), full(64, _NF),
    ]

    ap, pp, a1, p1, gai, gpij, d, v = pl.pallas_call(
        _edge_kernel,
        grid=grid,
        in_specs=in_specs,
        out_specs=out_specs,
        out_shape=out_shapes,
        compiler_params=pltpu.CompilerParams(
            dimension_semantics=("arbitrary", "arbitrary")),
        interpret=interpret,
    )(R, N.reshape(_B, AE, 1).astype(jnp.int32), NM.reshape(_B, AE, 1), a0, am,
      W_rbf, W_rbf.T, b_rbf.reshape(1, _NF),
      Wt1, bt1.reshape(1, _NF), Wt2, bt2.reshape(1, 64),
      Wt3.reshape(1, 64), bt3.reshape(1, 1), Wt1.T, Wt2.T,
      Wq1, bq1.reshape(1, _NF), Wq2, bq2.reshape(1, 64),
      Wq3.reshape(1, 64), bq3.reshape(1, 1), Wq1.T, Wq2.T)

    return (ap[..., 0], pp.reshape(_B, _A, _NB), a1,
            p1.reshape(_B, _A, _NB, _NF), gai,
            gpij.reshape(_B, _A, _NB), d.reshape(_B, _A, _NB),
            v.reshape(_B, _A, _NB, 3))


def kernel(R, Z, N, NM, emb, W_rbf, b_rbf, Wa1, ba1, Wa2, ba2,
           Wt1, bt1, Wt2, bt2, Wt3, bt3, Wq1, bq1, Wq2, bq2, Wq3, bq3):
    return _run(R, Z, N, NM, emb, W_rbf, b_rbf, Wa1, ba1, Wa2, ba2,
                Wt1, bt1, Wt2, bt2, Wt3, bt3, Wq1, bq1, Wq2, bq2, Wq3, bq3)


# speedup vs baseline: 5.4593x; 1.3709x over previous
import functools

import jax
import jax.numpy as jnp
from jax import lax
from jax.experimental import pallas as pl
from jax.experimental.pallas import tpu as pltpu

_B, _A, _NB = 8, 256, 64
_NF = 128
_RES = 20
_CUT = 5.0
_TA = 32
_E = _TA * _NB


def _silu_fb(y):
    s = jax.nn.sigmoid(y)
    return y * s, s * (1.0 + y * (1.0 - s))


def _prep_kernel(z_ref, emb_ref, wa1_ref, ba1_ref, wa2_ref, ba2_ref,
                 a0_ref, am_ref):
    z = z_ref[...]
    oh = (z == lax.broadcasted_iota(jnp.int32, (_B * _A, 16), 1)
          ).astype(jnp.float32)
    a0 = lax.dot(oh, emb_ref[...], preferred_element_type=jnp.float32,
                 precision=lax.Precision.HIGHEST)
    y = lax.dot(a0, wa1_ref[...], preferred_element_type=jnp.float32) + ba1_ref[...]
    h, _ = _silu_fb(y)
    am = lax.dot(h, wa2_ref[...], preferred_element_type=jnp.float32) + ba2_ref[...]
    a0_ref[...] = a0
    am_ref[...] = am


def _edge_kernel(r_ref, n_ref, nm_ref, a0_ref, am_ref,
                 chi_ref, cmid_ref, clo_ref, rhi_ref, rmid_ref, rlo_ref,
                 wrbf_ref, wrbft_ref, brbf_ref,
                 wt1_ref, bt1_ref, wt2_ref, bt2_ref, wt3t_ref, bt3_ref,
                 wt1t_ref, wt2t_ref,
                 wq1_ref, bq1_ref, wq2_ref, bq2_ref, wq3t_ref, bq3_ref,
                 wq1t_ref, wq2t_ref,
                 ap_ref, pp_ref, a1_ref, p1_ref, gai_ref, gp_ref, d_ref, v_ref):
    ia = pl.program_id(1)
    i0 = ia * _TA
    c = _CUT
    sq2c = (2.0 / c) ** 0.5

    nf = n_ref[0]
    nm = nm_ref[0]

    ohb = (nf == lax.broadcasted_iota(jnp.int32, (_E, _A), 1)
           ).astype(jnp.bfloat16)
    eidx = i0 + lax.broadcasted_iota(jnp.int32, (_E, 1), 0) // _NB
    self_e = nf == eidx

    rj = (lax.dot(ohb, rhi_ref[0], preferred_element_type=jnp.float32)
          + lax.dot(ohb, rmid_ref[0], preferred_element_type=jnp.float32)
          + lax.dot(ohb, rlo_ref[0], preferred_element_type=jnp.float32))
    ri = r_ref[0, pl.ds(i0, _TA), :]
    ri_e = jnp.broadcast_to(ri[:, None, :], (_TA, _NB, 3)).reshape(_E, 3)
    vv = jnp.where(self_e, 0.0, rj - ri_e)
    s = jnp.sum(vv * vv, axis=-1, keepdims=True) + 1e-12
    dv = jnp.sqrt(s)
    dd = dv * nm

    ddT = lax.transpose(dd, (1, 0))
    xT = ddT / c
    ltcT = (ddT < c).astype(jnp.float32)
    x8T = (xT * xT) * (xT * xT)
    x8T = x8T * x8T
    x9T = x8T * xT
    pcT = (1.0 - 55.0 * x9T + 99.0 * x9T * xT
           - 45.0 * x9T * xT * xT) * ltcT
    dpcT = (1.0 / c) * (-495.0 * x8T + 990.0 * x9T
                        - 495.0 * x9T * xT) * ltcT
    tT = jnp.tanh(10.0 * (5.0 - ddT))

    nrowT = (lax.broadcasted_iota(jnp.int32, (_RES, _E), 0) + 1
             ).astype(jnp.float32)
    argT = ddT * (jnp.pi / c) * nrowT
    dnT = ddT + 1e-8
    rdnT = 1.0 / dnT
    snT = jnp.sin(argT)
    csT = jnp.cos(argT)
    rbfT = sq2c * snT * rdnT
    drbfT = sq2c * rdnT * (nrowT * (jnp.pi / c) * csT - snT * rdnT)

    pc = lax.transpose(pcT, (1, 0))
    dpc = lax.transpose(dpcT, (1, 0))
    t = lax.transpose(tT, (1, 0))
    drbf = lax.transpose(drbfT, (1, 0))

    rbf_d = lax.dot_general(rbfT, wrbf_ref[...], (((0,), (0,)), ((), ())),
                            preferred_element_type=jnp.float32) + brbf_ref[...]
    rbf_m = rbf_d * pc

    a0i = a0_ref[0, pl.ds(i0, _TA), :]
    ami = am_ref[0, pl.ds(i0, _TA), :]
    a0i_e = jnp.broadcast_to(a0i[:, None, :], (_TA, _NB, _NF)).reshape(_E, _NF)
    ami_e = jnp.broadcast_to(ami[:, None, :], (_TA, _NB, _NF)).reshape(_E, _NF)
    gcat = (lax.dot(ohb, chi_ref[0], preferred_element_type=jnp.float32)
            + lax.dot(ohb, cmid_ref[0], preferred_element_type=jnp.float32)
            + lax.dot(ohb, clo_ref[0], preferred_element_type=jnp.float32))
    a0j = gcat[:, :_NF]
    amj = gcat[:, _NF:]

    am2 = ami_e * amj
    msij = am2 * rbf_m * nm
    p1v = a0i_e * a0j + msij
    a1v = a0i + jnp.sum(msij.reshape(_TA, _NB, _NF), axis=1)

    y1 = lax.dot(a1v, wt1_ref[...], preferred_element_type=jnp.float32) + bt1_ref[...]
    h1, d1 = _silu_fb(y1)
    y2 = lax.dot(h1, wt2_ref[...], preferred_element_type=jnp.float32) + bt2_ref[...]
    h2, d2 = _silu_fb(y2)
    wt3b = wt3t_ref[...].astype(jnp.bfloat16).astype(jnp.float32)
    h2b = h2.astype(jnp.bfloat16).astype(jnp.float32)
    apv = jnp.sum(h2b * wt3b, axis=-1, keepdims=True) + bt3_ref[...]
    g_y2 = wt3b * d2
    g_h1 = lax.dot(g_y2, wt2t_ref[...], preferred_element_type=jnp.float32)
    ga1 = lax.dot(g_h1 * d1, wt1t_ref[...],
                  preferred_element_type=jnp.float32)

    yq1 = lax.dot(p1v, wq1_ref[...], preferred_element_type=jnp.float32) + bq1_ref[...]
    q1, dq1 = _silu_fb(yq1)
    yq2 = lax.dot(q1, wq2_ref[...], preferred_element_type=jnp.float32) + bq2_ref[...]
    q2, dq2 = _silu_fb(yq2)
    wq3b = wq3t_ref[...].astype(jnp.bfloat16).astype(jnp.float32)
    q3 = jnp.sum(q2.astype(jnp.bfloat16).astype(jnp.float32) * wq3b,
                 axis=-1, keepdims=True) + bq3_ref[...]
    pp_sq = q3 * q3
    tap = 0.5 + 0.5 * t
    mask = (dd != 0.0).astype(jnp.float32)
    ppv = pp_sq * mask * tap

    g_q3 = 2.0 * q3 * mask * tap
    g_q3b = g_q3.astype(jnp.bfloat16).astype(jnp.float32)
    g_yq2 = (g_q3b * wq3b) * dq2
    g_q1 = lax.dot(g_yq2, wq2t_ref[...], preferred_element_type=jnp.float32)
    g_p1 = lax.dot(g_q1 * dq1, wq1t_ref[...],
                   preferred_element_type=jnp.float32)

    def chain_to_d(g_ms):
        g_rbf_m = g_ms * am2 * nm
        g_pc = jnp.sum(g_rbf_m * rbf_d, axis=-1, keepdims=True)
        g_rbf = lax.dot(g_rbf_m * pc, wrbft_ref[...],
                        preferred_element_type=jnp.float32)
        return (jnp.sum(g_rbf * drbf, axis=-1, keepdims=True)
                + g_pc * dpc)

    gd_pair = chain_to_d(g_p1) + pp_sq * mask * (-5.0 * (1.0 - t * t))

    ga1_e = jnp.broadcast_to(ga1[:, None, :], (_TA, _NB, _NF)).reshape(_E, _NF)
    gd_atom = chain_to_d(ga1_e)
    w = (gd_atom * nm / dv) * vv

    ap_ref[0] = apv
    pp_ref[0] = ppv
    a1_ref[0] = a1v
    p1_ref[0] = p1v
    gp_ref[0] = gd_pair
    d_ref[0] = dd
    v_ref[0] = vv

    cib = (eidx == lax.broadcasted_iota(jnp.int32, (_E, _A), 1)
           ).astype(jnp.bfloat16)
    ohci = ohb - cib
    whi = w.astype(jnp.bfloat16)
    wlo = (w - whi.astype(jnp.float32)).astype(jnp.bfloat16)
    scat = (lax.dot_general(ohci, whi, (((0,), (0,)), ((), ())),
                            preferred_element_type=jnp.float32)
            + lax.dot_general(ohci, wlo, (((0,), (0,)), ((), ())),
                              preferred_element_type=jnp.float32))

    @pl.when(ia == 0)
    def _():
        gai_ref[0] = jnp.zeros((_A, 3), dtype=jnp.float32)

    gai_ref[0] = gai_ref[0] + scat


@functools.partial(jax.jit, static_argnames=("interpret",))
def _run(R, Z, N, NM, emb, W_rbf, b_rbf, Wa1, ba1, Wa2, ba2,
         Wt1, bt1, Wt2, bt2, Wt3, bt3, Wq1, bq1, Wq2, bq2, Wq3, bq3,
         interpret=False):
    f32 = jnp.float32
    z2 = Z.reshape(_B * _A, 1).astype(jnp.int32)
    emb16 = jnp.zeros((16, _NF), f32).at[:10].set(emb)

    a0f, amf = pl.pallas_call(
        _prep_kernel,
        out_shape=(jax.ShapeDtypeStruct((_B * _A, _NF), f32),
                   jax.ShapeDtypeStruct((_B * _A, _NF), f32)),
        interpret=interpret,
    )(z2, emb16, Wa1, ba1.reshape(1, _NF), Wa2, ba2.reshape(1, _NF))

    a0 = a0f.reshape(_B, _A, _NF)
    am = amf.reshape(_B, _A, _NF)
    bf16 = jnp.bfloat16
    cat = jnp.concatenate([a0, am], axis=-1)
    chi = cat.astype(bf16)
    cr = cat - chi.astype(f32)
    cmid = cr.astype(bf16)
    clo = (cr - cmid.astype(f32)).astype(bf16)
    rhi = R.astype(bf16)
    rr = R - rhi.astype(f32)
    rmid = rr.astype(bf16)
    rlo = (rr - rmid.astype(f32)).astype(bf16)

    grid = (_B, _A // _TA)
    full = lambda *dims: pl.BlockSpec(dims, lambda b, i: (0,) * len(dims))
    batch = lambda *dims: pl.BlockSpec((1,) + dims, lambda b, i: (b,) + (0,) * len(dims))
    tile = lambda *dims: pl.BlockSpec((1, _TA) + dims,
                                      lambda b, i: (b, i) + (0,) * len(dims))

    AE = _A * _NB
    etile = lambda *dims: pl.BlockSpec((1, _E) + dims,
                                       lambda b, i: (b, i) + (0,) * len(dims))
    out_shapes = (
        jax.ShapeDtypeStruct((_B, _A, 1), f32),
        jax.ShapeDtypeStruct((_B, AE, 1), f32),
        jax.ShapeDtypeStruct((_B, _A, _NF), f32),
        jax.ShapeDtypeStruct((_B, AE, _NF), f32),
        jax.ShapeDtypeStruct((_B, _A, 3), f32),
        jax.ShapeDtypeStruct((_B, AE, 1), f32),
        jax.ShapeDtypeStruct((_B, AE, 1), f32),
        jax.ShapeDtypeStruct((_B, AE, 3), f32),
    )
    out_specs = (
        tile(1), etile(1), tile(_NF), etile(_NF),
        batch(_A, 3), etile(1), etile(1), etile(3),
    )
    in_specs = [
        batch(_A, 3),
        etile(1),
        etile(1),
        batch(_A, _NF),
        batch(_A, _NF),
        batch(_A, 2 * _NF),
        batch(_A, 2 * _NF),
        batch(_A, 2 * _NF),
        batch(_A, 3),
        batch(_A, 3),
        batch(_A, 3),
        full(_RES, _NF),
        full(_NF, _RES),
        full(1, _NF),
        full(_NF, _NF), full(1, _NF),
        full(_NF, 64), full(1, 64),
        full(1, 64), full(1, 1),
        full(_NF, _NF), full(64, _NF),
        full(_NF, _NF), full(1, _NF),
        full(_NF, 64), full(1, 64),
        full(1, 64), full(1, 1),
        full(_NF, _NF), full(64, _NF),
    ]

    ap, pp, a1, p1, gai, gpij, d, v = pl.pallas_call(
        _edge_kernel,
        grid=grid,
        in_specs=in_specs,
        out_specs=out_specs,
        out_shape=out_shapes,
        compiler_params=pltpu.CompilerParams(
            dimension_semantics=("arbitrary", "arbitrary")),
        interpret=interpret,
    )(R, N.reshape(_B, AE, 1).astype(jnp.int32), NM.reshape(_B, AE, 1), a0, am,
      chi, cmid, clo, rhi, rmid, rlo,
      W_rbf, W_rbf.T, b_rbf.reshape(1, _NF),
      Wt1, bt1.reshape(1, _NF), Wt2, bt2.reshape(1, 64),
      Wt3.reshape(1, 64), bt3.reshape(1, 1), Wt1.T, Wt2.T,
      Wq1, bq1.reshape(1, _NF), Wq2, bq2.reshape(1, 64),
      Wq3.reshape(1, 64), bq3.reshape(1, 1), Wq1.T, Wq2.T)

    return (ap[..., 0], pp.reshape(_B, _A, _NB), a1,
            p1.reshape(_B, _A, _NB, _NF), gai,
            gpij.reshape(_B, _A, _NB), d.reshape(_B, _A, _NB),
            v.reshape(_B, _A, _NB, 3))


def kernel(R, Z, N, NM, emb, W_rbf, b_rbf, Wa1, ba1, Wa2, ba2,
           Wt1, bt1, Wt2, bt2, Wt3, bt3, Wq1, bq1, Wq2, bq2, Wq3, bq3):
    return _run(R, Z, N, NM, emb, W_rbf, b_rbf, Wa1, ba1, Wa2, ba2,
                Wt1, bt1, Wt2, bt2, Wt3, bt3, Wq1, bq1, Wq2, bq2, Wq3, bq3)
